# baseline (speedup 1.0000x reference)
import dataclasses
import functools

import jax
import jax.numpy as jnp
from jax import lax
from jax.experimental import pallas as pl
from jax.experimental.pallas import tpu as pltpu
from jax.experimental.pallas import tpu_sc as plsc

XB = (-51.2, 51.2, 0.8)
YB = (-51.2, 51.2, 0.8)
ZB = (-10.0, 10.0, 20.0)
NX = 128
NY = 128
NCELL = NX * NY

B = 6
D = 41
C = 64
H = 32
W = 88
HW = H * W
BD = B * D

CH_PER_TILE = 4
NUM_SC = 2
NUM_PASSES = B // NUM_SC
LANES = 16
GROUPS = HW // LANES
DROWS_PER_TILE = (D + 15) // 16


def _sc_voxel_pool(x4, geom3):
    mesh = plsc.VectorSubcoreMesh(core_axis_name="c", subcore_axis_name="s")
    cp = pltpu.CompilerParams()
    if "needs_layout_passes" in pltpu.CompilerParams.__dataclass_fields__:
        cp = dataclasses.replace(cp, needs_layout_passes=False)

    @functools.partial(
        pl.kernel,
        compiler_params=cp,
        out_type=jax.ShapeDtypeStruct((B, C, NCELL), jnp.float32),
        mesh=mesh,
        scratch_types=(
            [pltpu.VMEM((NCELL,), jnp.float32) for _ in range(CH_PER_TILE)]
            + [
                pltpu.VMEM((2, CH_PER_TILE, HW), jnp.float32),
                pltpu.VMEM((2, HW), jnp.int32),
                pltpu.VMEM((HW * 3,), jnp.float32),
                pltpu.VMEM((HW,), jnp.int32),
                pltpu.VMEM_SHARED((D, HW), jnp.int32),
                pltpu.SemaphoreType.DMA,
                pltpu.SemaphoreType.DMA,
                pltpu.SemaphoreType.DMA,
                pltpu.SemaphoreType.DMA,
            ]
        ),
    )
    def sc_kernel(x_hbm, geom_hbm, out_hbm, g0, g1, g2, g3, xbuf, ibuf,
                  gbuf, irow, idxsp, sx0, sx1, si0, si1):
        grids = [g0, g1, g2, g3]
        sx = [sx0, sx1]
        si = [si0, si1]
        cid = lax.axis_index("c")
        sid = lax.axis_index("s")
        c0 = sid * CH_PER_TILE

        zero16 = jnp.zeros((LANES,), jnp.float32)
        off0 = lax.iota(jnp.int32, LANES) * 3
        off1 = off0 + 1
        off2 = off0 + 2

        def quantize_row(b, d):
            pltpu.sync_copy(geom_hbm.at[b, d], gbuf)

            @pl.loop(0, GROUPS, unroll=2)
            def _per_group(g):
                base3 = g * (LANES * 3)
                gx = plsc.load_gather(gbuf, [off0 + base3])
                gy = plsc.load_gather(gbuf, [off1 + base3])
                gz = plsc.load_gather(gbuf, [off2 + base3])
                keep = (
                    (gx >= XB[0]) & (gx < XB[1])
                    & (gy >= YB[0]) & (gy < YB[1])
                    & (gz >= ZB[0]) & (gz < ZB[1])
                )
                ix = ((gx - XB[0]) * (1.0 / XB[2])).astype(jnp.int32)
                iy = ((gy - YB[0]) * (1.0 / YB[2])).astype(jnp.int32)
                ix = jnp.minimum(jnp.maximum(ix, 0), NX - 1)
                iy = jnp.minimum(jnp.maximum(iy, 0), NY - 1)
                flat = ix * NY + iy
                irow[pl.ds(g * LANES, LANES)] = jnp.where(keep, flat, -1)

            pltpu.sync_copy(irow, idxsp.at[d])

        def start_d(b, d, s):
            pltpu.make_async_copy(
                x_hbm.at[b, d, pl.ds(c0, CH_PER_TILE)], xbuf.at[s], sx[s]
            ).start()
            pltpu.make_async_copy(idxsp.at[d], ibuf.at[s], si[s]).start()

        def wait_d(s):
            pltpu.make_async_copy(
                x_hbm.at[0, 0, pl.ds(0, CH_PER_TILE)], xbuf.at[s], sx[s]
            ).wait()
            pltpu.make_async_copy(idxsp.at[0], ibuf.at[s], si[s]).wait()

        def compute(s):
            @pl.loop(0, GROUPS, unroll=4)
            def _per_group(g):
                base = g * LANES
                idxv = ibuf[s, pl.ds(base, LANES)]
                mask = idxv >= 0
                for c in range(CH_PER_TILE):
                    vals = xbuf[s, c, pl.ds(base, LANES)]
                    plsc.addupdate_scatter(grids[c], [idxv], vals, mask=mask)

        for p in range(NUM_PASSES):
            b = p * NUM_SC + cid

            for r in range(DROWS_PER_TILE):
                d = sid + 16 * r
                if 16 * r + 15 < D:
                    quantize_row(b, d)
                else:
                    @pl.when(d < D)
                    def _():
                        quantize_row(b, d)

            @pl.loop(0, NCELL // LANES, unroll=8)
            def _zero(i):
                for c in range(CH_PER_TILE):
                    grids[c][pl.ds(i * LANES, LANES)] = zero16

            plsc.subcore_barrier()

            start_d(b, 0, 0)

            @pl.loop(0, (D - 1) // 2)
            def _per_pair(t):
                d = t * 2
                start_d(b, d + 1, 1)
                wait_d(0)
                compute(0)
                start_d(b, d + 2, 0)
                wait_d(1)
                compute(1)

            wait_d(0)
            compute(0)

            for c in range(CH_PER_TILE):
                pltpu.sync_copy(grids[c], out_hbm.at[b, c0 + c])

            if p + 1 < NUM_PASSES:
                plsc.subcore_barrier()

    return sc_kernel(x4, geom3)


@jax.jit
def kernel(geom_feats, x):
    geom3 = geom_feats.reshape(B, D, HW * 3)
    x4 = x.reshape(B, D, C, HW)
    out = _sc_voxel_pool(x4, geom3)
    return out.reshape(B, C, NX, NY)


# speedup vs baseline: 1.0138x; 1.0138x over previous
import dataclasses
import functools

import jax
import jax.numpy as jnp
from jax import lax
from jax.experimental import pallas as pl
from jax.experimental.pallas import tpu as pltpu
from jax.experimental.pallas import tpu_sc as plsc

XB = (-51.2, 51.2, 0.8)
YB = (-51.2, 51.2, 0.8)
ZB = (-10.0, 10.0, 20.0)
NX = 128
NY = 128
NCELL = NX * NY

B = 6
D = 41
C = 64
H = 32
W = 88
HW = H * W
BD = B * D

CH_PER_TILE = 4
NUM_SC = 2
NUM_PASSES = B // NUM_SC
LANES = 16
GROUPS = HW // LANES
DROWS_PER_TILE = (D + 15) // 16


def _sc_voxel_pool(x4, geom3):
    mesh = plsc.VectorSubcoreMesh(core_axis_name="c", subcore_axis_name="s")
    cp = pltpu.CompilerParams()
    if "needs_layout_passes" in pltpu.CompilerParams.__dataclass_fields__:
        cp = dataclasses.replace(cp, needs_layout_passes=False)

    @functools.partial(
        pl.kernel,
        compiler_params=cp,
        out_type=jax.ShapeDtypeStruct((NUM_SC, C, NCELL), jnp.float32),
        mesh=mesh,
        scratch_types=(
            [pltpu.VMEM((NCELL,), jnp.float32) for _ in range(CH_PER_TILE)]
            + [
                pltpu.VMEM((2, CH_PER_TILE, HW), jnp.float32),
                pltpu.VMEM((2, HW), jnp.int32),
                pltpu.VMEM((HW * 3,), jnp.float32),
                pltpu.VMEM((HW,), jnp.int32),
                pltpu.VMEM_SHARED((D, HW), jnp.int32),
                pltpu.SemaphoreType.DMA,
                pltpu.SemaphoreType.DMA,
                pltpu.SemaphoreType.DMA,
                pltpu.SemaphoreType.DMA,
            ]
        ),
    )
    def sc_kernel(x_hbm, geom_hbm, out_hbm, g0, g1, g2, g3, xbuf, ibuf,
                  gbuf, irow, idxsp, sx0, sx1, si0, si1):
        grids = [g0, g1, g2, g3]
        sx = [sx0, sx1]
        si = [si0, si1]
        cid = lax.axis_index("c")
        sid = lax.axis_index("s")
        c0 = sid * CH_PER_TILE

        zero16 = jnp.zeros((LANES,), jnp.float32)
        off0 = lax.iota(jnp.int32, LANES) * 3
        off1 = off0 + 1
        off2 = off0 + 2

        def quantize_row(b, d):
            pltpu.sync_copy(geom_hbm.at[b, d], gbuf)

            @pl.loop(0, GROUPS, unroll=2)
            def _per_group(g):
                base3 = g * (LANES * 3)
                gx = plsc.load_gather(gbuf, [off0 + base3])
                gy = plsc.load_gather(gbuf, [off1 + base3])
                gz = plsc.load_gather(gbuf, [off2 + base3])
                keep = (
                    (gx >= XB[0]) & (gx < XB[1])
                    & (gy >= YB[0]) & (gy < YB[1])
                    & (gz >= ZB[0]) & (gz < ZB[1])
                )
                ix = ((gx - XB[0]) * (1.0 / XB[2])).astype(jnp.int32)
                iy = ((gy - YB[0]) * (1.0 / YB[2])).astype(jnp.int32)
                ix = jnp.minimum(jnp.maximum(ix, 0), NX - 1)
                iy = jnp.minimum(jnp.maximum(iy, 0), NY - 1)
                flat = ix * NY + iy
                irow[pl.ds(g * LANES, LANES)] = jnp.where(keep, flat, -1)

            pltpu.sync_copy(irow, idxsp.at[d])

        def start_d(b, d, s):
            pltpu.make_async_copy(
                x_hbm.at[b, d, pl.ds(c0, CH_PER_TILE)], xbuf.at[s], sx[s]
            ).start()
            pltpu.make_async_copy(idxsp.at[d], ibuf.at[s], si[s]).start()

        def wait_d(s):
            pltpu.make_async_copy(
                x_hbm.at[0, 0, pl.ds(0, CH_PER_TILE)], xbuf.at[s], sx[s]
            ).wait()
            pltpu.make_async_copy(idxsp.at[0], ibuf.at[s], si[s]).wait()

        def compute(s):
            @pl.loop(0, GROUPS, unroll=4)
            def _per_group(g):
                base = g * LANES
                idxv = ibuf[s, pl.ds(base, LANES)]
                mask = idxv >= 0
                for c in range(CH_PER_TILE):
                    vals = xbuf[s, c, pl.ds(base, LANES)]
                    plsc.addupdate_scatter(grids[c], [idxv], vals, mask=mask)

        b = cid

        for r in range(DROWS_PER_TILE):
            d = sid + 16 * r
            if 16 * r + 15 < D:
                quantize_row(b, d)
            else:
                @pl.when(d < D)
                def _():
                    quantize_row(b, d)

        @pl.loop(0, NCELL // LANES, unroll=8)
        def _zero(i):
            for c in range(CH_PER_TILE):
                grids[c][pl.ds(i * LANES, LANES)] = zero16

        plsc.subcore_barrier()

        start_d(b, 0, 0)

        @pl.loop(0, (D - 1) // 2)
        def _per_pair(t):
            d = t * 2
            start_d(b, d + 1, 1)
            wait_d(0)
            compute(0)
            start_d(b, d + 2, 0)
            wait_d(1)
            compute(1)

        wait_d(0)
        compute(0)

        for c in range(CH_PER_TILE):
            pltpu.sync_copy(grids[c], out_hbm.at[b, c0 + c])

    return sc_kernel(x4, geom3)


@jax.jit
def kernel(geom_feats, x):
    outs = []
    for p in range(NUM_PASSES):
        sl = slice(p * NUM_SC, (p + 1) * NUM_SC)
        geom3 = geom_feats[sl].reshape(NUM_SC, D, HW * 3)
        x4 = x[sl].reshape(NUM_SC, D, C, HW)
        outs.append(_sc_voxel_pool(x4, geom3))
    out = jnp.concatenate(outs, axis=0)
    return out.reshape(B, C, NX, NY)


# speedup vs baseline: 1.0151x; 1.0013x over previous
import dataclasses
import functools

import jax
import jax.numpy as jnp
from jax import lax
from jax.experimental import pallas as pl
from jax.experimental.pallas import tpu as pltpu
from jax.experimental.pallas import tpu_sc as plsc

XB = (-51.2, 51.2, 0.8)
YB = (-51.2, 51.2, 0.8)
ZB = (-10.0, 10.0, 20.0)
NX = 128
NY = 128
NCELL = NX * NY

B = 6
D = 41
C = 64
H = 32
W = 88
HW = H * W
BD = B * D

CH_PER_TILE = 4
NUM_SC = 2
NUM_PASSES = B // NUM_SC
LANES = 16
GROUPS = HW // LANES
DROWS_PER_TILE = (D + 15) // 16


def _sc_voxel_pool(x4, geom3):
    mesh = plsc.VectorSubcoreMesh(core_axis_name="c", subcore_axis_name="s")
    cp = pltpu.CompilerParams()
    if "needs_layout_passes" in pltpu.CompilerParams.__dataclass_fields__:
        cp = dataclasses.replace(cp, needs_layout_passes=False)

    @functools.partial(
        pl.kernel,
        compiler_params=cp,
        out_type=jax.ShapeDtypeStruct((NUM_SC, C, NCELL), jnp.float32),
        mesh=mesh,
        scratch_types=(
            [pltpu.VMEM((NCELL,), jnp.float32) for _ in range(CH_PER_TILE)]
            + [
                pltpu.VMEM((2, CH_PER_TILE, HW), jnp.float32),
                pltpu.VMEM((2, HW), jnp.int32),
                pltpu.VMEM((HW * 3,), jnp.float32),
                pltpu.VMEM((HW,), jnp.int32),
                pltpu.VMEM_SHARED((D, HW), jnp.int32),
                pltpu.SemaphoreType.DMA,
                pltpu.SemaphoreType.DMA,
                pltpu.SemaphoreType.DMA,
                pltpu.SemaphoreType.DMA,
            ]
        ),
    )
    def sc_kernel(x_hbm, geom_hbm, out_hbm, g0, g1, g2, g3, xbuf, ibuf,
                  gbuf, irow, idxsp, sx0, sx1, si0, si1):
        grids = [g0, g1, g2, g3]
        sx = [sx0, sx1]
        si = [si0, si1]
        cid = lax.axis_index("c")
        sid = lax.axis_index("s")
        c0 = sid * CH_PER_TILE

        zero16 = jnp.zeros((LANES,), jnp.float32)
        off0 = lax.iota(jnp.int32, LANES) * 3
        off1 = off0 + 1
        off2 = off0 + 2

        def quantize_row(b, d):
            pltpu.sync_copy(geom_hbm.at[b, d], gbuf)

            @pl.loop(0, GROUPS, unroll=2)
            def _per_group(g):
                base3 = g * (LANES * 3)
                gx = plsc.load_gather(gbuf, [off0 + base3])
                gy = plsc.load_gather(gbuf, [off1 + base3])
                gz = plsc.load_gather(gbuf, [off2 + base3])
                keep = (
                    (gx >= XB[0]) & (gx < XB[1])
                    & (gy >= YB[0]) & (gy < YB[1])
                    & (gz >= ZB[0]) & (gz < ZB[1])
                )
                ix = ((gx - XB[0]) * (1.0 / XB[2])).astype(jnp.int32)
                iy = ((gy - YB[0]) * (1.0 / YB[2])).astype(jnp.int32)
                ix = jnp.minimum(jnp.maximum(ix, 0), NX - 1)
                iy = jnp.minimum(jnp.maximum(iy, 0), NY - 1)
                flat = ix * NY + iy
                irow[pl.ds(g * LANES, LANES)] = jnp.where(keep, flat, -1)

            pltpu.sync_copy(irow, idxsp.at[d])

        def start_d(b, d, s):
            pltpu.make_async_copy(
                x_hbm.at[b, d, pl.ds(c0, CH_PER_TILE)], xbuf.at[s], sx[s]
            ).start()
            pltpu.make_async_copy(idxsp.at[d], ibuf.at[s], si[s]).start()

        def wait_d(s):
            pltpu.make_async_copy(
                x_hbm.at[0, 0, pl.ds(0, CH_PER_TILE)], xbuf.at[s], sx[s]
            ).wait()
            pltpu.make_async_copy(idxsp.at[0], ibuf.at[s], si[s]).wait()

        def compute(s):
            @pl.loop(0, GROUPS, unroll=4)
            def _per_group(g):
                base = g * LANES
                idxv = ibuf[s, pl.ds(base, LANES)]
                mask = idxv >= 0
                for c in range(CH_PER_TILE):
                    vals = xbuf[s, c, pl.ds(base, LANES)]
                    plsc.addupdate_scatter(grids[c], [idxv], vals, mask=mask)

        b = cid

        for r in range(DROWS_PER_TILE):
            d = sid + 16 * r
            if 16 * r + 15 < D:
                quantize_row(b, d)
            else:
                @pl.when(d < D)
                def _():
                    quantize_row(b, d)

        @pl.loop(0, NCELL // LANES, unroll=8)
        def _zero(i):
            for c in range(CH_PER_TILE):
                grids[c][pl.ds(i * LANES, LANES)] = zero16

        plsc.subcore_barrier()

        start_d(b, 0, 0)

        @pl.loop(0, (D - 1) // 2)
        def _per_pair(t):
            d = t * 2
            start_d(b, d + 1, 1)
            wait_d(0)
            compute(0)
            start_d(b, d + 2, 0)
            wait_d(1)
            compute(1)

        wait_d(0)
        compute(0)

        for c in range(CH_PER_TILE):
            pltpu.sync_copy(grids[c], out_hbm.at[b, c0 + c])

    return sc_kernel(x4, geom3)


@jax.jit
def kernel(geom_feats, x):
    geom3_all = geom_feats.reshape(B, D, HW * 3)
    outs = []
    for p in range(NUM_PASSES):
        sl = slice(p * NUM_SC, (p + 1) * NUM_SC)
        x4 = x[sl].reshape(NUM_SC, D, C, HW)
        outs.append(_sc_voxel_pool(x4, geom3_all[sl]))
    out = jnp.concatenate(outs, axis=0)
    return out.reshape(B, C, NX, NY)
